# baseline (speedup 1.0000x reference)
import functools

import jax
import jax.numpy as jnp
from jax import lax
from jax.experimental import pallas as pl
from jax.experimental.pallas import tpu as pltpu
from jax.experimental.pallas import tpu_sc as plsc

_NCHUNK = 2


def _stage1_body(H, TAR, attn_ref, dec_ref, w_ref, b_ref, upd_ref):
  a = attn_ref[0]
  m = a[0:TAR, :]
  for h in range(1, H):
    m = m + a[h * TAR:(h + 1) * TAR, :]
  m = m * (1.0 / H)
  mmax = jnp.max(m, axis=-1, keepdims=True)
  e = jnp.exp(m - mmax)
  dist = e / jnp.sum(e, axis=-1, keepdims=True)
  x = jnp.dot(dec_ref[0], w_ref[...],
              preferred_element_type=jnp.float32) + b_ref[0, 0]
  pg = jax.nn.sigmoid(x)
  upd_ref[0] = (1.0 - pg) * dist


def _stage2_body(fin_ref, dec_ref, w_ref, b_ref, out_ref):
  x = jnp.dot(dec_ref[...], w_ref[...],
              preferred_element_type=jnp.float32) + b_ref[0, 0]
  pg = jax.nn.sigmoid(x)
  row = fin_ref[...]
  mmax = jnp.max(row, axis=-1, keepdims=True)
  e = jnp.exp(row - mmax)
  s = jnp.sum(e, axis=-1, keepdims=True)
  out_ref[...] = e * (pg / s)


def _stage4_body(probs_ref, prev_ref, out_ref):
  del prev_ref
  out_ref[0] = jnp.log(probs_ref[...])


def kernel(dec_output, final_output, attention_weights, encoder_input,
           W, b, inp_shape, tar_shape, training):
  B, TAR, D = dec_output.shape
  VOCAB = final_output.shape[-1]
  H = attention_weights.shape[1]
  INP = encoder_input.shape[1]
  R = B * TAR
  RC = R // _NCHUNK

  attn_r = attention_weights.reshape(B, H * TAR, INP)
  b2 = b.reshape(1, 1)
  dec2 = dec_output.reshape(R, D)
  fin2 = final_output.reshape(R, VOCAB)

  upd = pl.pallas_call(
      functools.partial(_stage1_body, H, TAR),
      grid=(B,),
      in_specs=[
          pl.BlockSpec((1, H * TAR, INP), lambda i: (i, 0, 0)),
          pl.BlockSpec((1, TAR, D), lambda i: (i, 0, 0)),
          pl.BlockSpec((D, 1), lambda i: (0, 0)),
          pl.BlockSpec((1, 1), lambda i: (0, 0)),
      ],
      out_specs=pl.BlockSpec((1, TAR, INP), lambda i: (i, 0, 0)),
      out_shape=jax.ShapeDtypeStruct((B, TAR, INP), jnp.float32),
  )(attn_r, dec_output, W, b2)
  upd2 = upd.reshape(R, INP)

  NC, NS = 2, 16
  NW = NC * NS
  RPW = RC // NW
  L = 16
  mesh = plsc.VectorSubcoreMesh(core_axis_name="c", subcore_axis_name="s")
  RB = 8

  def make_sc_chunk(c):
    @functools.partial(
        pl.kernel,
        out_type=(),
        mesh=mesh,
        compiler_params=pltpu.CompilerParams(needs_layout_passes=False),
        scratch_types=[
            pltpu.VMEM((VOCAB,), jnp.float32),
            pltpu.VMEM((INP,), jnp.int32),
            pltpu.VMEM((INP,), jnp.float32),
            pltpu.SemaphoreType.DMA,
        ],
    )
    def sc_scatter_add(enc_hbm, upd_hbm, probs_ref, row_v, idx_v, val_v, sem):
      w = lax.axis_index("s") * NC + lax.axis_index("c")
      bi = (c * RC + w * RPW) // TAR
      pltpu.sync_copy(enc_hbm.at[bi], idx_v)
      for j in range(RPW):
        r = w * RPW + j
        pltpu.sync_copy(probs_ref.at[r], row_v)
        pltpu.sync_copy(upd_hbm.at[c * RC + r], val_v)

        @pl.loop(0, INP // L)
        def _(k):
          iv = idx_v[pl.ds(k * L, L)]
          vv = val_v[pl.ds(k * L, L)]
          plsc.addupdate_scatter(row_v, [iv], vv)

        pltpu.sync_copy(row_v, probs_ref.at[r])

    return sc_scatter_add

  fixed = dict(dec=dec2, W=W, b2=b2)
  combined_chunks = []
  for c in range(_NCHUNK):
    probs_c = pl.pallas_call(
        _stage2_body,
        grid=(RC // RB,),
        in_specs=[
            pl.BlockSpec((RB, VOCAB), lambda j, c=c: (c * (RC // RB) + j, 0)),
            pl.BlockSpec((RB, D), lambda j, c=c: (c * (RC // RB) + j, 0)),
            pl.BlockSpec((D, 1), lambda j: (0, 0)),
            pl.BlockSpec((1, 1), lambda j: (0, 0)),
        ],
        out_specs=pl.BlockSpec((RB, VOCAB), lambda j: (j, 0)),
        out_shape=jax.ShapeDtypeStruct((RC, VOCAB), jnp.float32),
    )(fin2, fixed["dec"], fixed["W"], fixed["b2"])
    ref_c = jax.new_ref(probs_c)
    make_sc_chunk(c)(encoder_input, upd2, ref_c)
    combined_chunks.append(ref_c[...])

  BPC = RC // TAR
  TB = TAR // RB
  out = None
  for c in range(_NCHUNK):
    if out is None:
      args = (combined_chunks[c], jnp.zeros((1, 1), jnp.float32))
      prev_spec = pl.BlockSpec((1, 1), lambda j: (0, 0))
      aliases = {}
    else:
      args = (combined_chunks[c], out)
      prev_spec = pl.BlockSpec(memory_space=pl.ANY)
      aliases = {1: 0}
    out = pl.pallas_call(
        _stage4_body,
        grid=(RC // RB,),
        in_specs=[pl.BlockSpec((RB, VOCAB), lambda j: (j, 0)), prev_spec],
        out_specs=pl.BlockSpec(
            (1, RB, VOCAB),
            lambda j, c=c: (c * BPC + j // TB, j % TB, 0)),
        out_shape=jax.ShapeDtypeStruct((B, TAR, VOCAB), jnp.float32),
        input_output_aliases=aliases,
    )(*args)
  return out


# speedup vs baseline: 1.0656x; 1.0656x over previous
import functools

import jax
import jax.numpy as jnp
from jax import lax
from jax.experimental import pallas as pl
from jax.experimental.pallas import tpu as pltpu
from jax.experimental.pallas import tpu_sc as plsc

_NCHUNK = 4


def _stage1_body(H, TAR, attn_ref, dec_ref, w_ref, b_ref, upd_ref):
  a = attn_ref[0]
  m = a[0:TAR, :]
  for h in range(1, H):
    m = m + a[h * TAR:(h + 1) * TAR, :]
  m = m * (1.0 / H)
  mmax = jnp.max(m, axis=-1, keepdims=True)
  e = jnp.exp(m - mmax)
  dist = e / jnp.sum(e, axis=-1, keepdims=True)
  x = jnp.dot(dec_ref[0], w_ref[...],
              preferred_element_type=jnp.float32) + b_ref[0, 0]
  pg = jax.nn.sigmoid(x)
  upd_ref[0] = (1.0 - pg) * dist


def _stage2_body(fin_ref, dec_ref, w_ref, b_ref, out_ref):
  x = jnp.dot(dec_ref[...], w_ref[...],
              preferred_element_type=jnp.float32) + b_ref[0, 0]
  pg = jax.nn.sigmoid(x)
  row = fin_ref[...]
  mmax = jnp.max(row, axis=-1, keepdims=True)
  e = jnp.exp(row - mmax)
  s = jnp.sum(e, axis=-1, keepdims=True)
  out_ref[...] = e * (pg / s)


def _stage4_body(probs_ref, prev_ref, out_ref):
  del prev_ref
  out_ref[0] = jnp.log(probs_ref[...])


def kernel(dec_output, final_output, attention_weights, encoder_input,
           W, b, inp_shape, tar_shape, training):
  B, TAR, D = dec_output.shape
  VOCAB = final_output.shape[-1]
  H = attention_weights.shape[1]
  INP = encoder_input.shape[1]
  R = B * TAR
  RC = R // _NCHUNK

  attn_r = attention_weights.reshape(B, H * TAR, INP)
  b2 = b.reshape(1, 1)
  dec2 = dec_output.reshape(R, D)
  fin2 = final_output.reshape(R, VOCAB)

  upd = pl.pallas_call(
      functools.partial(_stage1_body, H, TAR),
      grid=(B,),
      in_specs=[
          pl.BlockSpec((1, H * TAR, INP), lambda i: (i, 0, 0)),
          pl.BlockSpec((1, TAR, D), lambda i: (i, 0, 0)),
          pl.BlockSpec((D, 1), lambda i: (0, 0)),
          pl.BlockSpec((1, 1), lambda i: (0, 0)),
      ],
      out_specs=pl.BlockSpec((1, TAR, INP), lambda i: (i, 0, 0)),
      out_shape=jax.ShapeDtypeStruct((B, TAR, INP), jnp.float32),
  )(attn_r, dec_output, W, b2)
  upd2 = upd.reshape(R, INP)

  NC, NS = 2, 16
  NW = NC * NS
  RPW = RC // NW
  L = 16
  mesh = plsc.VectorSubcoreMesh(core_axis_name="c", subcore_axis_name="s")
  RB = 16

  def make_sc_chunk(c):
    @functools.partial(
        pl.kernel,
        out_type=(),
        mesh=mesh,
        compiler_params=pltpu.CompilerParams(needs_layout_passes=False),
        scratch_types=[
            pltpu.VMEM((VOCAB,), jnp.float32),
            pltpu.VMEM((INP,), jnp.int32),
            pltpu.VMEM((INP,), jnp.float32),
            pltpu.SemaphoreType.DMA,
        ],
    )
    def sc_scatter_add(enc_hbm, upd_hbm, probs_ref, row_v, idx_v, val_v, sem):
      w = lax.axis_index("s") * NC + lax.axis_index("c")
      bi = (c * RC + w * RPW) // TAR
      pltpu.sync_copy(enc_hbm.at[bi], idx_v)
      for j in range(RPW):
        r = w * RPW + j
        pltpu.sync_copy(probs_ref.at[r], row_v)
        pltpu.sync_copy(upd_hbm.at[c * RC + r], val_v)

        @pl.loop(0, INP // L)
        def _(k):
          iv = idx_v[pl.ds(k * L, L)]
          vv = val_v[pl.ds(k * L, L)]
          plsc.addupdate_scatter(row_v, [iv], vv)

        pltpu.sync_copy(row_v, probs_ref.at[r])

    return sc_scatter_add

  fixed = dict(dec=dec2, W=W, b2=b2)
  combined_chunks = []
  for c in range(_NCHUNK):
    probs_c = pl.pallas_call(
        _stage2_body,
        grid=(RC // RB,),
        in_specs=[
            pl.BlockSpec((RB, VOCAB), lambda j, c=c: (c * (RC // RB) + j, 0)),
            pl.BlockSpec((RB, D), lambda j, c=c: (c * (RC // RB) + j, 0)),
            pl.BlockSpec((D, 1), lambda j: (0, 0)),
            pl.BlockSpec((1, 1), lambda j: (0, 0)),
        ],
        out_specs=pl.BlockSpec((RB, VOCAB), lambda j: (j, 0)),
        out_shape=jax.ShapeDtypeStruct((RC, VOCAB), jnp.float32),
    )(fin2, fixed["dec"], fixed["W"], fixed["b2"])
    ref_c = jax.new_ref(probs_c)
    make_sc_chunk(c)(encoder_input, upd2, ref_c)
    combined_chunks.append(ref_c[...])

  BPC = RC // TAR
  TB = TAR // RB
  out = None
  for c in range(_NCHUNK):
    if out is None:
      args = (combined_chunks[c], jnp.zeros((1, 1), jnp.float32))
      prev_spec = pl.BlockSpec((1, 1), lambda j: (0, 0))
      aliases = {}
    else:
      args = (combined_chunks[c], out)
      prev_spec = pl.BlockSpec(memory_space=pl.ANY)
      aliases = {1: 0}
    out = pl.pallas_call(
        _stage4_body,
        grid=(RC // RB,),
        in_specs=[pl.BlockSpec((RB, VOCAB), lambda j: (j, 0)), prev_spec],
        out_specs=pl.BlockSpec(
            (1, RB, VOCAB),
            lambda j, c=c: (c * BPC + j // TB, j % TB, 0)),
        out_shape=jax.ShapeDtypeStruct((B, TAR, VOCAB), jnp.float32),
        input_output_aliases=aliases,
    )(*args)
  return out


# speedup vs baseline: 1.0789x; 1.0125x over previous
import functools

import jax
import jax.numpy as jnp
from jax import lax
from jax.experimental import pallas as pl
from jax.experimental.pallas import tpu as pltpu
from jax.experimental.pallas import tpu_sc as plsc

_NCHUNK = 4


def _stage1_body(H, TAR, attn_ref, dec_ref, w_ref, b_ref, upd_ref):
  a = attn_ref[0]
  m = a[0:TAR, :]
  for h in range(1, H):
    m = m + a[h * TAR:(h + 1) * TAR, :]
  m = m * (1.0 / H)
  mmax = jnp.max(m, axis=-1, keepdims=True)
  e = jnp.exp(m - mmax)
  dist = e / jnp.sum(e, axis=-1, keepdims=True)
  x = jnp.dot(dec_ref[0], w_ref[...],
              preferred_element_type=jnp.float32) + b_ref[0, 0]
  pg = jax.nn.sigmoid(x)
  upd_ref[0] = (1.0 - pg) * dist


def _stage2_body(VOCAB, fin_ref, dec_ref, w_ref, b_ref, out_ref):
  x = jnp.dot(dec_ref[...], w_ref[...],
              preferred_element_type=jnp.float32) + b_ref[0, 0]
  pg = jax.nn.sigmoid(x)
  row = fin_ref[...]
  mmax = jnp.max(row, axis=-1, keepdims=True)
  e = jnp.exp(row - mmax)
  s = jnp.sum(e, axis=-1, keepdims=True)
  pad = out_ref.shape[-1] - VOCAB
  out_ref[:, pl.ds(0, VOCAB)] = e * (pg / s)
  out_ref[:, pl.ds(VOCAB, pad)] = jnp.ones((out_ref.shape[0], pad),
                                           jnp.float32)


def _stage4_body(VOCAB, probs_ref, prev_ref, out_ref):
  del prev_ref
  out_ref[0] = jnp.log(probs_ref[:, pl.ds(0, VOCAB)])


def kernel(dec_output, final_output, attention_weights, encoder_input,
           W, b, inp_shape, tar_shape, training):
  B, TAR, D = dec_output.shape
  VOCAB = final_output.shape[-1]
  H = attention_weights.shape[1]
  INP = encoder_input.shape[1]
  R = B * TAR
  RC = R // _NCHUNK

  attn_r = attention_weights.reshape(B, H * TAR, INP)
  b2 = b.reshape(1, 1)
  dec2 = dec_output.reshape(R, D)
  fin2 = final_output.reshape(R, VOCAB)

  upd = pl.pallas_call(
      functools.partial(_stage1_body, H, TAR),
      grid=(B,),
      in_specs=[
          pl.BlockSpec((1, H * TAR, INP), lambda i: (i, 0, 0)),
          pl.BlockSpec((1, TAR, D), lambda i: (i, 0, 0)),
          pl.BlockSpec((D, 1), lambda i: (0, 0)),
          pl.BlockSpec((1, 1), lambda i: (0, 0)),
      ],
      out_specs=pl.BlockSpec((1, TAR, INP), lambda i: (i, 0, 0)),
      out_shape=jax.ShapeDtypeStruct((B, TAR, INP), jnp.float32),
  )(attn_r, dec_output, W, b2)
  upd2 = upd.reshape(R, INP)

  NC, NS = 2, 16
  NW = NC * NS
  RPW = RC // NW
  L = 16
  mesh = plsc.VectorSubcoreMesh(core_axis_name="c", subcore_axis_name="s")
  RB = 16

  VOCABP = ((VOCAB + 127) // 128) * 128
  HV0 = VOCABP // 2
  HV1 = VOCABP - HV0
  T = 2 * RPW

  def make_sc_chunk(c):
    @functools.partial(
        pl.kernel,
        out_type=(),
        mesh=mesh,
        compiler_params=pltpu.CompilerParams(needs_layout_passes=False),
        scratch_types=[
            pltpu.VMEM((HV0,), jnp.float32),
            pltpu.VMEM((HV0,), jnp.float32),
            pltpu.VMEM((INP,), jnp.int32),
            pltpu.VMEM((RPW, INP), jnp.float32),
            pltpu.SemaphoreType.DMA,
            pltpu.SemaphoreType.DMA,
            pltpu.SemaphoreType.DMA,
            pltpu.SemaphoreType.DMA,
            pltpu.SemaphoreType.DMA,
        ],
    )
    def sc_scatter_add(enc_hbm, upd_hbm, probs_ref,
                       buf0, buf1, idx_v, val_v,
                       sin0, sin1, sout0, sout1, sup):
      w = lax.axis_index("s") * NC + lax.axis_index("c")
      bi = (c * RC + w * RPW) // TAR
      bufs = (buf0, buf1)
      sins = (sin0, sin1)
      souts = (sout0, sout1)
      pltpu.async_copy(upd_hbm.at[pl.ds(c * RC + w * RPW, RPW)], val_v,
                       sup).wait()
      pltpu.sync_copy(enc_hbm.at[bi], idx_v)

      def task(k):
        j, h = k // 2, k % 2
        base = h * HV0
        hv = HV0 if h == 0 else HV1
        return j, base, hv

      def start_in(k):
        j, base, hv = task(k)
        bb = bufs[k % 2]
        return pltpu.async_copy(
            probs_ref.at[w * RPW + j, pl.ds(base, hv)],
            bb.at[pl.ds(0, hv)], sins[k % 2])

      hin = {0: start_in(0)}
      hout = {}
      for k in range(T):
        if k + 1 < T:
          if k - 1 >= 0:
            hout[k - 1].wait()
          hin[k + 1] = start_in(k + 1)
        hin[k].wait()
        j, base, hv = task(k)
        bb = bufs[k % 2]

        @pl.loop(0, INP // L)
        def _(kk, base=base, hv=hv, bb=bb, j=j):
          iv = idx_v[pl.ds(kk * L, L)]
          vv = val_v[j, pl.ds(kk * L, L)]
          m = (iv >= base) & (iv < base + hv)
          li = jnp.where(m, iv - base, 0)
          plsc.addupdate_scatter(bb, [li], vv, mask=m)

        hout[k] = pltpu.async_copy(
            bb.at[pl.ds(0, hv)],
            probs_ref.at[w * RPW + j, pl.ds(base, hv)], souts[k % 2])
      hout[T - 2].wait()
      hout[T - 1].wait()

    return sc_scatter_add

  fixed = dict(dec=dec2, W=W, b2=b2)
  combined_chunks = []
  for c in range(_NCHUNK):
    probs_c = pl.pallas_call(
        functools.partial(_stage2_body, VOCAB),
        grid=(RC // RB,),
        in_specs=[
            pl.BlockSpec((RB, VOCAB), lambda j, c=c: (c * (RC // RB) + j, 0)),
            pl.BlockSpec((RB, D), lambda j, c=c: (c * (RC // RB) + j, 0)),
            pl.BlockSpec((D, 1), lambda j: (0, 0)),
            pl.BlockSpec((1, 1), lambda j: (0, 0)),
        ],
        out_specs=pl.BlockSpec((RB, VOCABP), lambda j: (j, 0)),
        out_shape=jax.ShapeDtypeStruct((RC, VOCABP), jnp.float32),
    )(fin2, fixed["dec"], fixed["W"], fixed["b2"])
    ref_c = jax.new_ref(probs_c)
    make_sc_chunk(c)(encoder_input, upd2, ref_c)
    combined_chunks.append(ref_c[...])

  BPC = RC // TAR
  TB = TAR // RB
  out = None
  for c in range(_NCHUNK):
    if out is None:
      args = (combined_chunks[c], jnp.zeros((1, 1), jnp.float32))
      prev_spec = pl.BlockSpec((1, 1), lambda j: (0, 0))
      aliases = {}
    else:
      args = (combined_chunks[c], out)
      prev_spec = pl.BlockSpec(memory_space=pl.ANY)
      aliases = {1: 0}
    out = pl.pallas_call(
        functools.partial(_stage4_body, VOCAB),
        grid=(RC // RB,),
        in_specs=[pl.BlockSpec((RB, VOCABP), lambda j: (j, 0)), prev_spec],
        out_specs=pl.BlockSpec(
            (1, RB, VOCAB),
            lambda j, c=c: (c * BPC + j // TB, j % TB, 0)),
        out_shape=jax.ShapeDtypeStruct((B, TAR, VOCAB), jnp.float32),
        input_output_aliases=aliases,
    )(*args)
  return out
